# baseline (speedup 1.0000x reference)
import functools

import jax
import jax.numpy as jnp
from jax import lax
from jax.experimental import pallas as pl
from jax.experimental.pallas import tpu as pltpu
from jax.experimental.pallas import tpu_sc as plsc

_NC = 2
_NS = 16
_NW = _NC * _NS
_L = 16
_CR = 8
_NIN = 4
_NOUT = 2


@functools.lru_cache(maxsize=None)
def _build(R, D):
    assert R % (_NW * _CR * _NIN) == 0 and D % _L == 0
    rows_per_w = R // _NW
    n_chunks = rows_per_w // _CR
    n_rounds = n_chunks // _NIN
    n_gran = D // _L

    mesh = plsc.VectorSubcoreMesh(core_axis_name="c", subcore_axis_name="s")

    @functools.partial(
        pl.kernel,
        out_type=jax.ShapeDtypeStruct((R, D), jnp.float32),
        mesh=mesh,
        scratch_types=[
            pltpu.VMEM((D,), jnp.int32),
            *([pltpu.VMEM((_CR, D), jnp.float32)] * _NIN),
            *([pltpu.VMEM((_CR, D), jnp.float32)] * _NOUT),
            *([pltpu.SemaphoreType.DMA] * (_NIN + _NOUT)),
        ],
        compiler_params=pltpu.CompilerParams(needs_layout_passes=False),
    )
    def permute(in_hbm, idx_hbm, out_hbm, idx_v, *bufs):
        ins = bufs[:_NIN]
        outs = bufs[_NIN:_NIN + _NOUT]
        isems = bufs[_NIN + _NOUT:2 * _NIN + _NOUT]
        osems = bufs[2 * _NIN + _NOUT:]

        wid = lax.axis_index("s") * _NC + lax.axis_index("c")
        base = wid * rows_per_w
        last_row0 = base + (n_chunks - 1) * _CR
        pltpu.sync_copy(idx_hbm, idx_v)

        lane = lax.iota(jnp.int32, _L)
        rvecs = [jnp.full((_L,), r, jnp.int32) for r in range(_CR)]

        def in_copy(row0, b):
            return pltpu.make_async_copy(
                in_hbm.at[pl.ds(row0, _CR)], ins[b], isems[b]
            )

        def out_copy(row0, b):
            return pltpu.make_async_copy(
                outs[b], out_hbm.at[pl.ds(row0, _CR)], osems[b]
            )

        for b in range(_NIN):
            in_copy(base + b * _CR, b).start()

        @pl.loop(0, n_rounds)
        def round_(t):
            for k in range(_NIN):
                ob = k % _NOUT
                row0 = base + (t * _NIN + k) * _CR
                in_copy(row0, k).wait()

                if k >= _NOUT:
                    out_copy(row0, ob).wait()
                else:
                    @pl.when(t > 0)
                    def _():
                        out_copy(row0, ob).wait()

                @plsc.parallel_loop(0, n_gran, unroll=4)
                def gran(j):
                    off = pl.multiple_of(j * _L, _L)
                    vidx = idx_v[pl.ds(off, _L)]
                    for r in range(_CR):
                        vals = plsc.load_gather(ins[k], [rvecs[r], vidx])
                        outs[ob][r, pl.ds(off, _L)] = vals

                out_copy(row0, ob).start()
                nxt = jnp.minimum(row0 + _NIN * _CR, last_row0)
                in_copy(nxt, k).start()

        for b in range(_NIN):
            in_copy(last_row0, b).wait()
        for b in range(_NOUT):
            out_copy(last_row0, b).wait()

    return permute


def kernel(inputs, idxs):
    shape = inputs.shape
    D = shape[-1]
    x = inputs.reshape(-1, D)
    out = _build(x.shape[0], D)(x, idxs)
    return out.reshape(shape)
